# baseline (speedup 1.0000x reference)
import jax
import jax.numpy as jnp
from jax.experimental import pallas as pl
from jax.experimental.pallas import tpu as pltpu

KS = 3
PAD = 1
GAMMA = 1.0
C_IN = 32
C_OUT = 32
H_IMG = 32
W_IMG = 32
HP = C_IN * KS * KS
D2 = 2.0 * HP
L = H_IMG * W_IMG
LANES = 128
NT = L // LANES
MAX_ITERS = 640
OU = 4
N_PRE = 7


def _unfold_kernel(xp_ref, u_ref):
    xv = xp_ref[0]
    for ij in range(KS * KS):
        i, j = divmod(ij, KS)
        u_ref[0, ij] = xv[:, i:i + H_IMG, j:j + W_IMG]


def _make_solver(ms, mxs, include_neg):
    def update(m, t):
        act = m > t
        k = jnp.sum(jnp.where(act, 1.0, 0.0), axis=0, keepdims=True)
        s = jnp.sum(jnp.where(act, m, 0.0), axis=0, keepdims=True)
        if include_neg:
            nm = -m
            actn = nm > t
            k = k + jnp.sum(jnp.where(actn, 1.0, 0.0), axis=0, keepdims=True)
            s = s + jnp.sum(jnp.where(actn, nm, 0.0), axis=0, keepdims=True)
        return jnp.maximum(t, (s - GAMMA) / k)

    def body(carry):
        i, ts, _ = carry
        ts1 = tuple(update(m, t) for m, t in zip(ms, ts))
        deltas = [t1 - t for t1, t in zip(ts1, ts)]
        dmax = deltas[0]
        for d in deltas[1:]:
            dmax = jnp.maximum(dmax, d)
        prog = jnp.max(dmax) > 0.0
        ts2 = tuple(update(m, t) for m, t in zip(ms, ts1))
        return i + 1, ts2, prog

    def cond(carry):
        i, _, prog = carry
        return jnp.logical_and(i < MAX_ITERS, prog)

    def run():
        ts = tuple(mx - GAMMA for mx in mxs)
        for _ in range(N_PRE):
            ts = tuple(update(m, t) for m, t in zip(ms, ts))
        init = (jnp.int32(0), ts, jnp.bool_(True))
        _, ts, _ = jax.lax.while_loop(cond, body, init)
        return ts

    return run


def _mp_kernel(a_ref, w_ref, o_ref):
    a = a_ref[0]
    ms, mxs = [], []
    for s in range(OU):
        c = w_ref[s, :, 0:1]
        for sgn in (1.0, -1.0):
            m = jnp.abs(a + sgn * c)
            ms.append(m)
            mxs.append(jnp.max(m, axis=0, keepdims=True))
    mn = mxs[0]
    for mx in mxs[1:]:
        mn = jnp.minimum(mn, mx)
    fast_ok = jnp.min(mn) >= GAMMA
    taus = jax.lax.cond(
        fast_ok,
        _make_solver(ms, mxs, include_neg=False),
        _make_solver(ms, mxs, include_neg=True),
    )
    for s in range(OU):
        res = (0.5 * GAMMA * D2) * (taus[2 * s] - taus[2 * s + 1])
        o_ref[0, s] = res.reshape(1, 1, LANES)


def kernel(x, weight):
    b_n = x.shape[0]
    xp = jnp.pad(x, ((0, 0), (0, 0), (PAD, PAD), (PAD, PAD)))
    unfolded = pl.pallas_call(
        _unfold_kernel,
        grid=(b_n,),
        in_specs=[pl.BlockSpec((1, C_IN, H_IMG + 2 * PAD, W_IMG + 2 * PAD),
                               lambda b: (b, 0, 0, 0))],
        out_specs=pl.BlockSpec((1, KS * KS, C_IN, H_IMG, W_IMG),
                               lambda b: (b, 0, 0, 0, 0)),
        out_shape=jax.ShapeDtypeStruct((b_n, KS * KS, C_IN, H_IMG, W_IMG),
                                       jnp.float32),
        compiler_params=pltpu.CompilerParams(
            dimension_semantics=("parallel",)),
        name="mpconv_unfold",
    )(xp)
    patches = unfolded.reshape(b_n, HP, L)
    w_rows = weight.transpose(2, 3, 1, 0).reshape(HP, C_OUT)
    w3 = jnp.broadcast_to(w_rows.T[:, :, None], (C_OUT, HP, 8))
    out = pl.pallas_call(
        _mp_kernel,
        grid=(b_n, NT, C_OUT // OU),
        in_specs=[
            pl.BlockSpec((1, HP, LANES), lambda b, t, o: (b, 0, t)),
            pl.BlockSpec((OU, HP, 8), lambda b, t, o: (o, 0, 0)),
        ],
        out_specs=pl.BlockSpec((1, OU, 1, 1, LANES),
                               lambda b, t, o: (b, o, t, 0, 0)),
        out_shape=jax.ShapeDtypeStruct((b_n, C_OUT, NT, 1, LANES),
                                       jnp.float32),
        compiler_params=pltpu.CompilerParams(
            dimension_semantics=("parallel", "parallel", "arbitrary")),
        name="mpconv_mp",
    )(patches, w3)
    return out.reshape(b_n, C_OUT, H_IMG, W_IMG)


# speedup vs baseline: 1.0380x; 1.0380x over previous
import jax
import jax.numpy as jnp
from jax.experimental import pallas as pl
from jax.experimental.pallas import tpu as pltpu

KS = 3
PAD = 1
GAMMA = 1.0
C_IN = 32
C_OUT = 32
H_IMG = 32
W_IMG = 32
HP = C_IN * KS * KS
D2 = 2.0 * HP
L = H_IMG * W_IMG
LANES = 128
NT = L // LANES
MAX_ITERS = 640
OU = 4
N_PRE = 3


def _unfold_kernel(xp_ref, u_ref):
    xv = xp_ref[0]
    for ij in range(KS * KS):
        i, j = divmod(ij, KS)
        u_ref[0, ij] = xv[:, i:i + H_IMG, j:j + W_IMG]


def _make_solver(ms, mxs, include_neg):
    def update(m, t):
        act = m > t
        k = jnp.sum(jnp.where(act, 1.0, 0.0), axis=0, keepdims=True)
        s = jnp.sum(jnp.where(act, m, 0.0), axis=0, keepdims=True)
        if include_neg:
            nm = -m
            actn = nm > t
            k = k + jnp.sum(jnp.where(actn, 1.0, 0.0), axis=0, keepdims=True)
            s = s + jnp.sum(jnp.where(actn, nm, 0.0), axis=0, keepdims=True)
        return jnp.maximum(t, (s - GAMMA) / k)

    def body(carry):
        i, ts, _ = carry
        ts1 = tuple(update(m, t) for m, t in zip(ms, ts))
        deltas = [t1 - t for t1, t in zip(ts1, ts)]
        dmax = deltas[0]
        for d in deltas[1:]:
            dmax = jnp.maximum(dmax, d)
        prog = jnp.max(dmax) > 0.0
        ts2 = tuple(update(m, t) for m, t in zip(ms, ts1))
        return i + 1, ts2, prog

    def cond(carry):
        i, _, prog = carry
        return jnp.logical_and(i < MAX_ITERS, prog)

    def run():
        ts = tuple(mx - GAMMA for mx in mxs)
        for _ in range(N_PRE):
            ts = tuple(update(m, t) for m, t in zip(ms, ts))
        init = (jnp.int32(0), ts, jnp.bool_(True))
        _, ts, _ = jax.lax.while_loop(cond, body, init)
        return ts

    return run


def _mp_kernel(a_ref, w_ref, o_ref):
    a = a_ref[0]
    ms, mxs = [], []
    for s in range(OU):
        c = w_ref[s, :, 0:1]
        for sgn in (1.0, -1.0):
            m = jnp.abs(a + sgn * c)
            ms.append(m)
            mxs.append(jnp.max(m, axis=0, keepdims=True))
    mn = mxs[0]
    for mx in mxs[1:]:
        mn = jnp.minimum(mn, mx)
    fast_ok = jnp.min(mn) >= GAMMA
    taus = jax.lax.cond(
        fast_ok,
        _make_solver(ms, mxs, include_neg=False),
        _make_solver(ms, mxs, include_neg=True),
    )
    for s in range(OU):
        res = (0.5 * GAMMA * D2) * (taus[2 * s] - taus[2 * s + 1])
        o_ref[0, s] = res.reshape(1, 1, LANES)


def kernel(x, weight):
    b_n = x.shape[0]
    xp = jnp.pad(x, ((0, 0), (0, 0), (PAD, PAD), (PAD, PAD)))
    unfolded = pl.pallas_call(
        _unfold_kernel,
        grid=(b_n,),
        in_specs=[pl.BlockSpec((1, C_IN, H_IMG + 2 * PAD, W_IMG + 2 * PAD),
                               lambda b: (b, 0, 0, 0))],
        out_specs=pl.BlockSpec((1, KS * KS, C_IN, H_IMG, W_IMG),
                               lambda b: (b, 0, 0, 0, 0)),
        out_shape=jax.ShapeDtypeStruct((b_n, KS * KS, C_IN, H_IMG, W_IMG),
                                       jnp.float32),
        compiler_params=pltpu.CompilerParams(
            dimension_semantics=("parallel",)),
        name="mpconv_unfold",
    )(xp)
    patches = unfolded.reshape(b_n, HP, L)
    w_rows = weight.transpose(2, 3, 1, 0).reshape(HP, C_OUT)
    w3 = jnp.broadcast_to(w_rows.T[:, :, None], (C_OUT, HP, 8))
    out = pl.pallas_call(
        _mp_kernel,
        grid=(b_n, NT, C_OUT // OU),
        in_specs=[
            pl.BlockSpec((1, HP, LANES), lambda b, t, o: (b, 0, t)),
            pl.BlockSpec((OU, HP, 8), lambda b, t, o: (o, 0, 0)),
        ],
        out_specs=pl.BlockSpec((1, OU, 1, 1, LANES),
                               lambda b, t, o: (b, o, t, 0, 0)),
        out_shape=jax.ShapeDtypeStruct((b_n, C_OUT, NT, 1, LANES),
                                       jnp.float32),
        compiler_params=pltpu.CompilerParams(
            dimension_semantics=("parallel", "parallel", "arbitrary")),
        name="mpconv_mp",
    )(patches, w3)
    return out.reshape(b_n, C_OUT, H_IMG, W_IMG)


# speedup vs baseline: 1.0605x; 1.0217x over previous
import jax
import jax.numpy as jnp
from jax.experimental import pallas as pl
from jax.experimental.pallas import tpu as pltpu

KS = 3
PAD = 1
GAMMA = 1.0
C_IN = 32
C_OUT = 32
H_IMG = 32
W_IMG = 32
HP = C_IN * KS * KS
D2 = 2.0 * HP
L = H_IMG * W_IMG
LANES = 256
NT = L // LANES
MAX_ITERS = 640
OU = 2
N_PRE = 5


def _unfold_kernel(xp_ref, u_ref):
    xv = xp_ref[0]
    for ij in range(KS * KS):
        i, j = divmod(ij, KS)
        u_ref[0, ij] = xv[:, i:i + H_IMG, j:j + W_IMG]


def _make_solver(ms, mxs, include_neg):
    def update(m, t):
        act = m > t
        k = jnp.sum(jnp.where(act, 1.0, 0.0), axis=0, keepdims=True)
        s = jnp.sum(jnp.where(act, m, 0.0), axis=0, keepdims=True)
        if include_neg:
            nm = -m
            actn = nm > t
            k = k + jnp.sum(jnp.where(actn, 1.0, 0.0), axis=0, keepdims=True)
            s = s + jnp.sum(jnp.where(actn, nm, 0.0), axis=0, keepdims=True)
        return jnp.maximum(t, (s - GAMMA) / k)

    def body(carry):
        i, ts, _ = carry
        ts1 = tuple(update(m, t) for m, t in zip(ms, ts))
        deltas = [t1 - t for t1, t in zip(ts1, ts)]
        dmax = deltas[0]
        for d in deltas[1:]:
            dmax = jnp.maximum(dmax, d)
        prog = jnp.max(dmax) > 0.0
        ts2 = tuple(update(m, t) for m, t in zip(ms, ts1))
        return i + 1, ts2, prog

    def cond(carry):
        i, _, prog = carry
        return jnp.logical_and(i < MAX_ITERS, prog)

    def run():
        ts = tuple(mx - GAMMA for mx in mxs)
        for _ in range(N_PRE):
            ts = tuple(update(m, t) for m, t in zip(ms, ts))
        init = (jnp.int32(0), ts, jnp.bool_(True))
        _, ts, _ = jax.lax.while_loop(cond, body, init)
        return ts

    return run


def _mp_kernel(a_ref, w_ref, o_ref):
    a = a_ref[0]
    ms, mxs = [], []
    for s in range(OU):
        c = w_ref[s, :, 0:1]
        for sgn in (1.0, -1.0):
            m = jnp.abs(a + sgn * c)
            ms.append(m)
            mxs.append(jnp.max(m, axis=0, keepdims=True))
    mn = mxs[0]
    for mx in mxs[1:]:
        mn = jnp.minimum(mn, mx)
    fast_ok = jnp.min(mn) >= GAMMA
    taus = jax.lax.cond(
        fast_ok,
        _make_solver(ms, mxs, include_neg=False),
        _make_solver(ms, mxs, include_neg=True),
    )
    for s in range(OU):
        res = (0.5 * GAMMA * D2) * (taus[2 * s] - taus[2 * s + 1])
        o_ref[0, s] = res.reshape(1, 1, LANES)


def kernel(x, weight):
    b_n = x.shape[0]
    xp = jnp.pad(x, ((0, 0), (0, 0), (PAD, PAD), (PAD, PAD)))
    unfolded = pl.pallas_call(
        _unfold_kernel,
        grid=(b_n,),
        in_specs=[pl.BlockSpec((1, C_IN, H_IMG + 2 * PAD, W_IMG + 2 * PAD),
                               lambda b: (b, 0, 0, 0))],
        out_specs=pl.BlockSpec((1, KS * KS, C_IN, H_IMG, W_IMG),
                               lambda b: (b, 0, 0, 0, 0)),
        out_shape=jax.ShapeDtypeStruct((b_n, KS * KS, C_IN, H_IMG, W_IMG),
                                       jnp.float32),
        compiler_params=pltpu.CompilerParams(
            dimension_semantics=("parallel",)),
        name="mpconv_unfold",
    )(xp)
    patches = unfolded.reshape(b_n, HP, L)
    w_rows = weight.transpose(2, 3, 1, 0).reshape(HP, C_OUT)
    w3 = jnp.broadcast_to(w_rows.T[:, :, None], (C_OUT, HP, 8))
    out = pl.pallas_call(
        _mp_kernel,
        grid=(b_n, NT, C_OUT // OU),
        in_specs=[
            pl.BlockSpec((1, HP, LANES), lambda b, t, o: (b, 0, t)),
            pl.BlockSpec((OU, HP, 8), lambda b, t, o: (o, 0, 0)),
        ],
        out_specs=pl.BlockSpec((1, OU, 1, 1, LANES),
                               lambda b, t, o: (b, o, t, 0, 0)),
        out_shape=jax.ShapeDtypeStruct((b_n, C_OUT, NT, 1, LANES),
                                       jnp.float32),
        compiler_params=pltpu.CompilerParams(
            dimension_semantics=("parallel", "parallel", "arbitrary")),
        name="mpconv_mp",
    )(patches, w3)
    return out.reshape(b_n, C_OUT, H_IMG, W_IMG)


# speedup vs baseline: 1.0823x; 1.0206x over previous
import jax
import jax.numpy as jnp
from jax.experimental import pallas as pl
from jax.experimental.pallas import tpu as pltpu

KS = 3
PAD = 1
GAMMA = 1.0
C_IN = 32
C_OUT = 32
H_IMG = 32
W_IMG = 32
HP = C_IN * KS * KS
D2 = 2.0 * HP
L = H_IMG * W_IMG
LANES = 256
NT = L // LANES
MAX_ITERS = 640
OU = 4
N_PRE = 5


def _unfold_kernel(xp_ref, u_ref):
    xv = xp_ref[0]
    for ij in range(KS * KS):
        i, j = divmod(ij, KS)
        u_ref[0, ij] = xv[:, i:i + H_IMG, j:j + W_IMG]


def _make_solver(ms, mxs, include_neg):
    def update(m, t):
        act = m > t
        k = jnp.sum(jnp.where(act, 1.0, 0.0), axis=0, keepdims=True)
        s = jnp.sum(jnp.where(act, m, 0.0), axis=0, keepdims=True)
        if include_neg:
            nm = -m
            actn = nm > t
            k = k + jnp.sum(jnp.where(actn, 1.0, 0.0), axis=0, keepdims=True)
            s = s + jnp.sum(jnp.where(actn, nm, 0.0), axis=0, keepdims=True)
        return jnp.maximum(t, (s - GAMMA) / k)

    def body(carry):
        i, ts, _ = carry
        ts1 = tuple(update(m, t) for m, t in zip(ms, ts))
        deltas = [t1 - t for t1, t in zip(ts1, ts)]
        dmax = deltas[0]
        for d in deltas[1:]:
            dmax = jnp.maximum(dmax, d)
        prog = jnp.max(dmax) > 0.0
        ts2 = tuple(update(m, t) for m, t in zip(ms, ts1))
        return i + 1, ts2, prog

    def cond(carry):
        i, _, prog = carry
        return jnp.logical_and(i < MAX_ITERS, prog)

    def run():
        ts = tuple(mx - GAMMA for mx in mxs)
        for _ in range(N_PRE):
            ts = tuple(update(m, t) for m, t in zip(ms, ts))
        init = (jnp.int32(0), ts, jnp.bool_(True))
        _, ts, _ = jax.lax.while_loop(cond, body, init)
        return ts

    return run


def _mp_kernel(a_ref, w_ref, o_ref):
    a = a_ref[0]
    ms, mxs = [], []
    for s in range(OU):
        c = w_ref[s, :, 0:1]
        for sgn in (1.0, -1.0):
            m = jnp.abs(a + sgn * c)
            ms.append(m)
            mxs.append(jnp.max(m, axis=0, keepdims=True))
    mn = mxs[0]
    for mx in mxs[1:]:
        mn = jnp.minimum(mn, mx)
    fast_ok = jnp.min(mn) >= GAMMA
    taus = jax.lax.cond(
        fast_ok,
        _make_solver(ms, mxs, include_neg=False),
        _make_solver(ms, mxs, include_neg=True),
    )
    for s in range(OU):
        res = (0.5 * GAMMA * D2) * (taus[2 * s] - taus[2 * s + 1])
        o_ref[0, s] = res.reshape(1, 1, LANES)


def kernel(x, weight):
    b_n = x.shape[0]
    xp = jnp.pad(x, ((0, 0), (0, 0), (PAD, PAD), (PAD, PAD)))
    unfolded = pl.pallas_call(
        _unfold_kernel,
        grid=(b_n,),
        in_specs=[pl.BlockSpec((1, C_IN, H_IMG + 2 * PAD, W_IMG + 2 * PAD),
                               lambda b: (b, 0, 0, 0))],
        out_specs=pl.BlockSpec((1, KS * KS, C_IN, H_IMG, W_IMG),
                               lambda b: (b, 0, 0, 0, 0)),
        out_shape=jax.ShapeDtypeStruct((b_n, KS * KS, C_IN, H_IMG, W_IMG),
                                       jnp.float32),
        compiler_params=pltpu.CompilerParams(
            dimension_semantics=("parallel",)),
        name="mpconv_unfold",
    )(xp)
    patches = unfolded.reshape(b_n, HP, L)
    w_rows = weight.transpose(2, 3, 1, 0).reshape(HP, C_OUT)
    w3 = jnp.broadcast_to(w_rows.T[:, :, None], (C_OUT, HP, 8))
    out = pl.pallas_call(
        _mp_kernel,
        grid=(b_n, NT, C_OUT // OU),
        in_specs=[
            pl.BlockSpec((1, HP, LANES), lambda b, t, o: (b, 0, t)),
            pl.BlockSpec((OU, HP, 8), lambda b, t, o: (o, 0, 0)),
        ],
        out_specs=pl.BlockSpec((1, OU, 1, 1, LANES),
                               lambda b, t, o: (b, o, t, 0, 0)),
        out_shape=jax.ShapeDtypeStruct((b_n, C_OUT, NT, 1, LANES),
                                       jnp.float32),
        compiler_params=pltpu.CompilerParams(
            dimension_semantics=("parallel", "parallel", "arbitrary")),
        name="mpconv_mp",
    )(patches, w3)
    return out.reshape(b_n, C_OUT, H_IMG, W_IMG)
